# baseline (speedup 1.0000x reference)
import functools

import jax
import jax.numpy as jnp
from jax import lax
from jax.experimental import pallas as pl
from jax.experimental.pallas import tpu as pltpu
from jax.experimental.pallas import tpu_sc as plsc

_LANES = 16
_CHUNK = 128


@functools.partial(jax.jit, static_argnames=("batch", "dim"))
def _run(user_table, item_table, users2d, items2d, *, batch, dim):
    info = plsc.get_sparse_core_info()
    n_workers = info.num_cores * info.num_subcores
    b_per_w = batch // n_workers
    n_chunks = b_per_w // _CHUNK
    n_groups = b_per_w // _LANES

    mesh = plsc.VectorSubcoreMesh(core_axis_name="c", subcore_axis_name="s")

    @functools.partial(
        pl.kernel,
        out_type=jax.ShapeDtypeStruct((batch,), jnp.float32),
        mesh=mesh,
        scratch_types=[
            pltpu.VMEM((n_chunks, _CHUNK), jnp.int32),
            pltpu.VMEM((n_chunks, _CHUNK), jnp.int32),
            pltpu.VMEM((b_per_w, dim), jnp.float32),
            pltpu.VMEM((b_per_w, dim), jnp.float32),
            pltpu.VMEM((b_per_w,), jnp.float32),
            pltpu.SemaphoreType.DMA,
        ],
        compiler_params=pltpu.CompilerParams(
            needs_layout_passes=False, use_tc_tiling_on_sc=False),
    )
    def sc_kernel(user_hbm, item_hbm, users_hbm, items_hbm, out_hbm,
                  uidx_v, iidx_v, urows_v, irows_v, scores_v, sem):
        wid = lax.axis_index("s") * info.num_cores + lax.axis_index("c")
        idx_row0 = wid * n_chunks

        pltpu.sync_copy(users_hbm.at[pl.ds(idx_row0, n_chunks)], uidx_v)
        pltpu.sync_copy(items_hbm.at[pl.ds(idx_row0, n_chunks)], iidx_v)

        copies = []
        for j in range(n_chunks):
            dst = pl.ds(j * _CHUNK, _CHUNK)
            copies.append(
                pltpu.async_copy(user_hbm.at[uidx_v.at[j]], urows_v.at[dst], sem))
            copies.append(
                pltpu.async_copy(item_hbm.at[iidx_v.at[j]], irows_v.at[dst], sem))
        for c in copies:
            c.wait()

        lanes = lax.iota(jnp.int32, _LANES)

        def group_body(g, carry):
            base = pl.multiple_of(g * _LANES, _LANES)
            vec = jnp.zeros((_LANES,), jnp.float32)
            for k in range(_LANES):
                u = urows_v[base + k, :]
                i = irows_v[base + k, :]
                s = jnp.sum(u * i)
                vec = jnp.where(lanes == k, s, vec)
            scores_v[pl.ds(base, _LANES)] = vec
            return carry

        lax.fori_loop(0, n_groups, group_body, 0)

        pltpu.sync_copy(scores_v, out_hbm.at[pl.ds(wid * b_per_w, b_per_w)])

    return sc_kernel(user_table, item_table, users2d, items2d)


def kernel(user_table, item_table, users, items):
    batch = users.shape[0]
    dim = user_table.shape[1]
    users2d = users.astype(jnp.int32).reshape(batch // _CHUNK, _CHUNK)
    items2d = items.astype(jnp.int32).reshape(batch // _CHUNK, _CHUNK)
    return _run(user_table, item_table, users2d, items2d,
                batch=batch, dim=dim)
